# Initial kernel scaffold; baseline (speedup 1.0000x reference)
import jax
import jax.numpy as jnp
from jax.experimental import pallas as pl


def kernel(x, edge_index, edge_label_index, W1, b1, W2, b2, Wlin, blin):
    raise NotImplementedError("write your pallas kernel here")


# speedup vs baseline: 9.0772x; 9.0772x over previous
import functools

import jax
import jax.numpy as jnp
from jax import lax
from jax.experimental import pallas as pl
from jax.experimental.pallas import tpu as pltpu
from jax.experimental.pallas import tpu_sc as plsc

_N = 10000
_D = 128
_H = 128
_E = 320000
_L = 20000

_NC = 2
_NS = 16
_NW = _NC * _NS
_EPT = _E // _NW
_K = 80
_S = _EPT // _K
_NP = 10240
_RPT = _NP // _NS

_LP = 20480
_KL = 128
_SL = (_LP // _NW) // _KL

_mesh = plsc.VectorSubcoreMesh(core_axis_name="c", subcore_axis_name="s")


def _deg_body(dst_hbm, ones_hbm, z128_hbm, out_hbm, dstv, onesv, accd):
    c = lax.axis_index("c")
    s = lax.axis_index("s")
    wid = c * _NS + s
    pltpu.sync_copy(z128_hbm.at[pl.ds(s * _RPT, _RPT)],
                    accd.at[pl.ds(s * _RPT, _RPT)])
    pltpu.sync_copy(ones_hbm, onesv)
    plsc.subcore_barrier()

    def step(i, carry):
        base = wid * _EPT + i * _K
        pltpu.sync_copy(dst_hbm.at[pl.ds(base, _K)], dstv)
        pltpu.sync_copy(onesv, accd.at[dstv], add=True)
        return carry

    lax.fori_loop(0, _S, step, 0)
    plsc.subcore_barrier()
    pltpu.sync_copy(accd.at[pl.ds(s * _RPT, _RPT)],
                    out_hbm.at[c, pl.ds(s * _RPT, _RPT)])


_sc_deg = functools.partial(
    pl.kernel,
    out_type=jax.ShapeDtypeStruct((_NC, _NP, _H), jnp.float32),
    mesh=_mesh,
    scratch_types=[
        pltpu.VMEM((_K,), jnp.int32),
        pltpu.VMEM((_K, _H), jnp.float32),
        pltpu.VMEM_SHARED((_NP, _H), jnp.float32),
    ],
)(_deg_body)


def _spmm_body(hs_hbm, src_hbm, dst_hbm, z128_hbm, out_hbm,
               srcv, dstv, rows, acc, sem):
    c = lax.axis_index("c")
    s = lax.axis_index("s")
    wid = c * _NS + s
    pltpu.sync_copy(z128_hbm.at[pl.ds(s * _RPT, _RPT)],
                    acc.at[pl.ds(s * _RPT, _RPT)])
    plsc.subcore_barrier()

    def step(i, carry):
        base = wid * _EPT + i * _K
        pltpu.sync_copy(src_hbm.at[pl.ds(base, _K)], srcv)
        pltpu.async_copy(hs_hbm.at[srcv], rows, sem).wait()
        pltpu.sync_copy(dst_hbm.at[pl.ds(base, _K)], dstv)
        pltpu.sync_copy(rows, acc.at[dstv], add=True)
        return carry

    lax.fori_loop(0, _S, step, 0)
    plsc.subcore_barrier()
    pltpu.sync_copy(acc.at[pl.ds(s * _RPT, _RPT)],
                    out_hbm.at[c, pl.ds(s * _RPT, _RPT)])


_sc_spmm = functools.partial(
    pl.kernel,
    out_type=jax.ShapeDtypeStruct((_NC, _NP, _H), jnp.float32),
    mesh=_mesh,
    scratch_types=[
        pltpu.VMEM((_K,), jnp.int32),
        pltpu.VMEM((_K,), jnp.int32),
        pltpu.VMEM((_K, _H), jnp.float32),
        pltpu.VMEM_SHARED((_NP, _H), jnp.float32),
        pltpu.SemaphoreType.DMA,
    ],
)(_spmm_body)


def _dec_body(z_hbm, s_hbm, d_hbm, zs_out, zd_out, idxv, rowsbuf, sem):
    c = lax.axis_index("c")
    s = lax.axis_index("s")
    wid = c * _NS + s

    def step(i, carry):
        base = wid * (_SL * _KL) + i * _KL
        pltpu.sync_copy(s_hbm.at[pl.ds(base, _KL)], idxv)
        pltpu.async_copy(z_hbm.at[idxv], rowsbuf, sem).wait()
        pltpu.sync_copy(rowsbuf, zs_out.at[pl.ds(base, _KL)])
        pltpu.sync_copy(d_hbm.at[pl.ds(base, _KL)], idxv)
        pltpu.async_copy(z_hbm.at[idxv], rowsbuf, sem).wait()
        pltpu.sync_copy(rowsbuf, zd_out.at[pl.ds(base, _KL)])
        return carry

    lax.fori_loop(0, _SL, step, 0)


_sc_dec = functools.partial(
    pl.kernel,
    out_type=(jax.ShapeDtypeStruct((_LP, _H), jnp.float32),
              jax.ShapeDtypeStruct((_LP, _H), jnp.float32)),
    mesh=_mesh,
    scratch_types=[
        pltpu.VMEM((_KL,), jnp.int32),
        pltpu.VMEM((_KL, _H), jnp.float32),
        pltpu.SemaphoreType.DMA,
    ],
)(_dec_body)


def _dinv_from(degp_ref):
    deg = degp_ref[0][:_N] + degp_ref[1][:_N]
    d0 = deg[:, 0:1]
    return jnp.where(d0 > 0, lax.rsqrt(jnp.maximum(d0, 1e-12)), 0.0)


def _tcA_body(degp_ref, x_ref, w1_ref, hs1_ref):
    dinv = _dinv_from(degp_ref)
    h = jnp.dot(x_ref[...], w1_ref[...], preferred_element_type=jnp.float32)
    hs1_ref[...] = dinv * h


def _tcB_body(degp_ref, acc_ref, b1_ref, w2_ref, h1_ref, hs2_ref):
    dinv = _dinv_from(degp_ref)
    h1 = jnp.tanh(dinv * (acc_ref[0][:_N] + acc_ref[1][:_N]) + b1_ref[...])
    h1_ref[...] = h1
    hs2_ref[...] = dinv * jnp.dot(h1, w2_ref[...],
                                  preferred_element_type=jnp.float32)


def _tcC_body(degp_ref, acc_ref, b2_ref, h1_ref, wlin_ref, blin_ref, z_ref):
    dinv = _dinv_from(degp_ref)
    h2 = jnp.tanh(dinv * (acc_ref[0][:_N] + acc_ref[1][:_N]) + b2_ref[...])
    zz = jnp.maximum(h1_ref[...], h2)
    z_ref[...] = jnp.dot(zz, wlin_ref[...],
                         preferred_element_type=jnp.float32) + blin_ref[...]


def _tcD_body(zs_ref, zd_ref, out_ref):
    out_ref[...] = jnp.sum(zs_ref[...] * zd_ref[...], axis=1, keepdims=True)


_tcA = pl.pallas_call(_tcA_body, out_shape=jax.ShapeDtypeStruct((_N, _H), jnp.float32))
_tcB = pl.pallas_call(_tcB_body, out_shape=(jax.ShapeDtypeStruct((_N, _H), jnp.float32),
                                            jax.ShapeDtypeStruct((_N, _H), jnp.float32)))
_tcC = pl.pallas_call(_tcC_body, out_shape=jax.ShapeDtypeStruct((_N, _H), jnp.float32))
_tcD = pl.pallas_call(_tcD_body, out_shape=jax.ShapeDtypeStruct((_LP, 1), jnp.float32))


def kernel(x, edge_index, edge_label_index, W1, b1, W2, b2, Wlin, blin):
    src = edge_index[0]
    dst = edge_index[1]
    z128 = jnp.zeros((_NP, _H), jnp.float32)
    ones128 = jnp.ones((_K, _H), jnp.float32)

    degp = _sc_deg(dst, ones128, z128)
    hs1 = _tcA(degp, x, W1)
    accA = _sc_spmm(hs1, src, dst, z128)
    h1, hs2 = _tcB(degp, accA, b1.reshape(1, _H), W2)
    accB = _sc_spmm(hs2, src, dst, z128)
    z = _tcC(degp, accB, b2.reshape(1, _H), h1, Wlin, blin.reshape(1, _H))

    pad = jnp.zeros((_LP - _L,), jnp.int32)
    sl = jnp.concatenate([edge_label_index[0], pad])
    dl = jnp.concatenate([edge_label_index[1], pad])
    zs, zd = _sc_dec(z, sl, dl)
    prod = _tcD(zs, zd)
    return prod[:_L, 0]


# speedup vs baseline: 18.1522x; 1.9998x over previous
import functools

import jax
import jax.numpy as jnp
from jax import lax
from jax.experimental import pallas as pl
from jax.experimental.pallas import tpu as pltpu
from jax.experimental.pallas import tpu_sc as plsc

_N = 10000
_D = 128
_H = 128
_E = 320000
_L = 20000

_NC = 2
_NS = 16
_NW = _NC * _NS
_EPT = _E // _NW
_K = 100
_S = _EPT // _K
_NP = 10240
_RPT = _NP // _NS

_LP = 20480
_KL = 128
_SL = (_LP // _NW) // _KL

_mesh = plsc.VectorSubcoreMesh(core_axis_name="c", subcore_axis_name="s")


def _deg_body(dst3_hbm, ones_hbm, z128_hbm, out_hbm, dstall, onesv, accd):
    c = lax.axis_index("c")
    s = lax.axis_index("s")
    wid = c * _NS + s
    pltpu.sync_copy(z128_hbm.at[pl.ds(s * _RPT, _RPT)],
                    accd.at[pl.ds(s * _RPT, _RPT)])
    pltpu.sync_copy(ones_hbm, onesv)
    pltpu.sync_copy(dst3_hbm.at[wid], dstall)
    plsc.subcore_barrier()

    def step(i, carry):
        pltpu.sync_copy(onesv, accd.at[dstall.at[i]], add=True)
        return carry

    lax.fori_loop(0, _S, step, 0)
    plsc.subcore_barrier()
    pltpu.sync_copy(accd.at[pl.ds(s * _RPT, _RPT)],
                    out_hbm.at[c, pl.ds(s * _RPT, _RPT)])


_sc_deg = functools.partial(
    pl.kernel,
    out_type=jax.ShapeDtypeStruct((_NC, _NP, _H), jnp.float32),
    mesh=_mesh,
    scratch_types=[
        pltpu.VMEM((_S, _K), jnp.int32),
        pltpu.VMEM((_K, _H), jnp.float32),
        pltpu.VMEM_SHARED((_NP, _H), jnp.float32),
    ],
)(_deg_body)


_B = 2


def _spmm_body(hs_hbm, src3_hbm, dst3_hbm, z128_hbm, out_hbm,
               srcv, dstall, rows, acc, *sems):
    sem_g = sems[:2]
    sem_i = sems[2:]
    c = lax.axis_index("c")
    s = lax.axis_index("s")
    wid = c * _NS + s
    pltpu.sync_copy(z128_hbm.at[pl.ds(s * _RPT, _RPT)],
                    acc.at[pl.ds(s * _RPT, _RPT)])
    pltpu.sync_copy(dst3_hbm.at[wid], dstall)
    plsc.subcore_barrier()

    def wait_idx(jj, q):
        pltpu.make_async_copy(src3_hbm.at[wid, jj], srcv.at[q],
                              sem_i[q]).wait()

    def wait_gather(b):
        pltpu.make_async_copy(hs_hbm.at[srcv.at[0]], rows.at[b],
                              sem_g[b]).wait()

    for q in range(4):
        pltpu.async_copy(src3_hbm.at[wid, q], srcv.at[q], sem_i[q])
    for b in range(2):
        wait_idx(b, b)
        pltpu.async_copy(hs_hbm.at[srcv.at[b]], rows.at[b], sem_g[b])

    def group(g, carry):
        for t in range(4):
            j = 4 * g + t
            b = t % 2
            qn = (t + 2) % 4
            wait_gather(b)
            pltpu.sync_copy(rows.at[b], acc.at[dstall.at[j]], add=True)
            wait_idx(j + 2, qn)
            pltpu.async_copy(hs_hbm.at[srcv.at[qn]], rows.at[b], sem_g[b])
            pltpu.async_copy(src3_hbm.at[wid, j + 4], srcv.at[t], sem_i[t])
        return carry

    lax.fori_loop(0, _S // 4 - 1, group, 0)

    for t in range(4):
        j = (_S - 4) + t
        b = t % 2
        qn = (t + 2) % 4
        wait_gather(b)
        pltpu.sync_copy(rows.at[b], acc.at[dstall.at[j]], add=True)
        if j + 2 <= _S - 1:
            wait_idx(j + 2, qn)
            pltpu.async_copy(hs_hbm.at[srcv.at[qn]], rows.at[b], sem_g[b])

    plsc.subcore_barrier()
    pltpu.sync_copy(acc.at[pl.ds(s * _RPT, _RPT)],
                    out_hbm.at[c, pl.ds(s * _RPT, _RPT)])


_sc_spmm = functools.partial(
    pl.kernel,
    out_type=jax.ShapeDtypeStruct((_NC, _NP, _H), jnp.float32),
    mesh=_mesh,
    scratch_types=[
        pltpu.VMEM((4, _K), jnp.int32),
        pltpu.VMEM((_S, _K), jnp.int32),
        pltpu.VMEM((2, _K, _H), jnp.float32),
        pltpu.VMEM_SHARED((_NP, _H), jnp.float32),
    ] + [pltpu.SemaphoreType.DMA] * 6,
)(_spmm_body)


def _dec_body(z_hbm, s3_hbm, d3_hbm, zs_out, zd_out, sall, dall,
              zsb, zdb, sem_s0, sem_s1, sem_d0, sem_d1):
    c = lax.axis_index("c")
    s = lax.axis_index("s")
    wid = c * _NS + s
    sem_s = (sem_s0, sem_s1)
    sem_d = (sem_d0, sem_d1)
    pltpu.sync_copy(s3_hbm.at[wid], sall)
    pltpu.sync_copy(d3_hbm.at[wid], dall)
    pltpu.async_copy(z_hbm.at[sall.at[0]], zsb.at[0], sem_s[0])
    pltpu.async_copy(z_hbm.at[dall.at[0]], zdb.at[0], sem_d[0])
    for i in range(_SL):
        p = i % 2
        if i + 1 < _SL:
            q = (i + 1) % 2
            pltpu.async_copy(z_hbm.at[sall.at[i + 1]], zsb.at[q], sem_s[q])
            pltpu.async_copy(z_hbm.at[dall.at[i + 1]], zdb.at[q], sem_d[q])
        base = wid * (_SL * _KL) + i * _KL
        pltpu.make_async_copy(z_hbm.at[sall.at[i]], zsb.at[p], sem_s[p]).wait()
        pltpu.sync_copy(zsb.at[p], zs_out.at[pl.ds(base, _KL)])
        pltpu.make_async_copy(z_hbm.at[dall.at[i]], zdb.at[p], sem_d[p]).wait()
        pltpu.sync_copy(zdb.at[p], zd_out.at[pl.ds(base, _KL)])


_sc_dec = functools.partial(
    pl.kernel,
    out_type=(jax.ShapeDtypeStruct((_LP, _H), jnp.float32),
              jax.ShapeDtypeStruct((_LP, _H), jnp.float32)),
    mesh=_mesh,
    scratch_types=[
        pltpu.VMEM((_SL, _KL), jnp.int32),
        pltpu.VMEM((_SL, _KL), jnp.int32),
        pltpu.VMEM((2, _KL, _H), jnp.float32),
        pltpu.VMEM((2, _KL, _H), jnp.float32),
        pltpu.SemaphoreType.DMA,
        pltpu.SemaphoreType.DMA,
        pltpu.SemaphoreType.DMA,
        pltpu.SemaphoreType.DMA,
    ],
)(_dec_body)


def _dinv_from(degp_ref):
    deg = degp_ref[0][:_N] + degp_ref[1][:_N]
    d0 = deg[:, 0:1]
    return jnp.where(d0 > 0, lax.rsqrt(jnp.maximum(d0, 1e-12)), 0.0)


def _tcA_body(degp_ref, x_ref, w1_ref, hs1_ref):
    dinv = _dinv_from(degp_ref)
    h = jnp.dot(x_ref[...], w1_ref[...], preferred_element_type=jnp.float32)
    hs1_ref[...] = dinv * h


def _tcB_body(degp_ref, acc_ref, b1_ref, w2_ref, h1_ref, hs2_ref):
    dinv = _dinv_from(degp_ref)
    h1 = jnp.tanh(dinv * (acc_ref[0][:_N] + acc_ref[1][:_N]) + b1_ref[...])
    h1_ref[...] = h1
    hs2_ref[...] = dinv * jnp.dot(h1, w2_ref[...],
                                  preferred_element_type=jnp.float32)


def _tcC_body(degp_ref, acc_ref, b2_ref, h1_ref, wlin_ref, blin_ref, z_ref):
    dinv = _dinv_from(degp_ref)
    h2 = jnp.tanh(dinv * (acc_ref[0][:_N] + acc_ref[1][:_N]) + b2_ref[...])
    zz = jnp.maximum(h1_ref[...], h2)
    z_ref[...] = jnp.dot(zz, wlin_ref[...],
                         preferred_element_type=jnp.float32) + blin_ref[...]


def _tcD_body(zs_ref, zd_ref, out_ref):
    out_ref[...] = jnp.sum(zs_ref[...] * zd_ref[...], axis=1, keepdims=True)


_tcA = pl.pallas_call(_tcA_body, out_shape=jax.ShapeDtypeStruct((_N, _H), jnp.float32))
_tcB = pl.pallas_call(_tcB_body, out_shape=(jax.ShapeDtypeStruct((_N, _H), jnp.float32),
                                            jax.ShapeDtypeStruct((_N, _H), jnp.float32)))
_tcC = pl.pallas_call(_tcC_body, out_shape=jax.ShapeDtypeStruct((_N, _H), jnp.float32))
_tcD = pl.pallas_call(_tcD_body, out_shape=jax.ShapeDtypeStruct((_LP, 1), jnp.float32))


def kernel(x, edge_index, edge_label_index, W1, b1, W2, b2, Wlin, blin):
    src3 = edge_index[0].reshape(_NW, _S, _K)
    dst3 = edge_index[1].reshape(_NW, _S, _K)
    z128 = jnp.zeros((_NP, _H), jnp.float32)
    ones128 = jnp.ones((_K, _H), jnp.float32)

    degp = _sc_deg(dst3, ones128, z128)
    hs1 = _tcA(degp, x, W1)
    accA = _sc_spmm(hs1, src3, dst3, z128)
    h1, hs2 = _tcB(degp, accA, b1.reshape(1, _H), W2)
    accB = _sc_spmm(hs2, src3, dst3, z128)
    z = _tcC(degp, accB, b2.reshape(1, _H), h1, Wlin, blin.reshape(1, _H))

    pad = jnp.zeros((2, _LP - _L), jnp.int32)
    eli = jnp.concatenate([edge_label_index, pad], axis=1)
    s3 = eli[0].reshape(_NW, _SL, _KL)
    d3 = eli[1].reshape(_NW, _SL, _KL)
    zs, zd = _sc_dec(z, s3, d3)
    prod = _tcD(zs, zd)
    return prod[:_L, 0]
